# baseline (speedup 1.0000x reference)
import functools

import jax
import jax.numpy as jnp
from jax import lax
from jax.experimental import pallas as pl
from jax.experimental.pallas import tpu as pltpu
from jax.experimental.pallas import tpu_sc as plsc

VOCAB = 1000000
EMB = 64
B = 16384
C = 20
TOTAL = B + B * C
NC, NS = 2, 16
NW = NC * NS
CHUNK = 128
PER_W = TOTAL // NW
NCHUNK = PER_W // CHUNK
assert NCHUNK * CHUNK * NW == TOTAL


def _gather_all(table, idx2d):
    mesh = plsc.VectorSubcoreMesh(core_axis_name="c", subcore_axis_name="s",
                                  num_cores=NC, num_subcores=NS)

    @functools.partial(
        pl.kernel,
        out_type=jax.ShapeDtypeStruct((TOTAL, EMB), jnp.float32),
        mesh=mesh,
        scratch_types=[
            pltpu.VMEM((NCHUNK, CHUNK), jnp.int32),
            pltpu.VMEM((CHUNK, EMB), jnp.float32),
            pltpu.SemaphoreType.DMA,
        ],
        compiler_params=pltpu.CompilerParams(use_tc_tiling_on_sc=False),
    )
    def k(table_hbm, idx_hbm, out_hbm, idx_v, buf, sem):
        wid = lax.axis_index("s") * NC + lax.axis_index("c")
        base = wid * NCHUNK
        pltpu.sync_copy(idx_hbm.at[wid], idx_v)

        @pl.loop(0, NCHUNK)
        def _(j):
            pltpu.async_copy(table_hbm.at[idx_v.at[j]], buf, sem).wait()
            pltpu.sync_copy(buf, out_hbm.at[pl.ds((base + j) * CHUNK, CHUNK)])

    return k(table, idx2d)


def kernel(center_ids, context_ids, W_center, W_context):
    idx = jnp.concatenate(
        [center_ids.astype(jnp.int32), context_ids.reshape(-1).astype(jnp.int32)]
    ).reshape(NW, NCHUNK, CHUNK)
    rows = _gather_all(W_center, idx)
    embs_center = rows[:B][:, :, None]
    embs_context = rows[B:].reshape(B, C, EMB)
    return (embs_center, embs_context)


# speedup vs baseline: 1.3681x; 1.3681x over previous
import functools

import jax
import jax.numpy as jnp
from jax import lax
from jax.experimental import pallas as pl
from jax.experimental.pallas import tpu as pltpu
from jax.experimental.pallas import tpu_sc as plsc

VOCAB = 1000000
EMB = 64
B = 16384
C = 20
NC, NS = 2, 16
NW = NC * NS
CHUNK = 128
CCH = B // NW // CHUNK
XCH = B * C // NW // CHUNK
NCHUNK = CCH + XCH
NBUF = 4


def _gather_all(table, idx3d):
    mesh = plsc.VectorSubcoreMesh(core_axis_name="c", subcore_axis_name="s",
                                  num_cores=NC, num_subcores=NS)

    @functools.partial(
        pl.kernel,
        out_type=(
            jax.ShapeDtypeStruct((B, EMB), jnp.float32),
            jax.ShapeDtypeStruct((B * C, EMB), jnp.float32),
        ),
        mesh=mesh,
        scratch_types=[
            pltpu.VMEM((NCHUNK, CHUNK), jnp.int32),
            pltpu.VMEM((NBUF, CHUNK, EMB), jnp.float32),
        ] + [pltpu.SemaphoreType.DMA] * (2 * NBUF),
        compiler_params=pltpu.CompilerParams(use_tc_tiling_on_sc=False),
    )
    def k(table_hbm, idx_hbm, outc_hbm, outx_hbm, idx_v, buf, *sems):
        gsem, wsem = sems[:NBUF], sems[NBUF:]
        wid = lax.axis_index("s") * NC + lax.axis_index("c")
        cbase = wid * CCH * CHUNK
        xbase = wid * XCH * CHUNK
        pltpu.sync_copy(idx_hbm.at[wid], idx_v)

        def issue_gather(j, b):
            pltpu.async_copy(table_hbm.at[idx_v.at[j]], buf.at[b], gsem[b])

        def wait_gather(j, b):
            pltpu.make_async_copy(table_hbm.at[idx_v.at[j]], buf.at[b],
                                  gsem[b]).wait()

        def issue_wb(j, b):
            @pl.when(j < CCH)
            def _():
                pltpu.async_copy(buf.at[b],
                                 outc_hbm.at[pl.ds(cbase + j * CHUNK, CHUNK)],
                                 wsem[b])

            @pl.when(j >= CCH)
            def _():
                pltpu.async_copy(
                    buf.at[b],
                    outx_hbm.at[pl.ds(xbase + (j - CCH) * CHUNK, CHUNK)],
                    wsem[b])

        def wait_wb(b):
            pltpu.make_async_copy(buf.at[b], outx_hbm.at[pl.ds(0, CHUNK)],
                                  wsem[b]).wait()

        for b in range(NBUF):
            issue_gather(b, b)

        @pl.loop(0, NCHUNK - NBUF, step=NBUF)
        def _(j0):
            for b in range(NBUF):
                j = j0 + b
                wait_gather(j, b)
                issue_wb(j, b)
                wait_wb(b)
                issue_gather(j + NBUF, b)

        for b in range(NBUF):
            j = NCHUNK - NBUF + b
            wait_gather(j, b)
            issue_wb(j, b)
            wait_wb(b)

    return k(table, idx3d)


def kernel(center_ids, context_ids, W_center, W_context):
    idx = jnp.concatenate(
        [
            center_ids.astype(jnp.int32).reshape(NW, CCH * CHUNK),
            context_ids.astype(jnp.int32).reshape(NW, XCH * CHUNK),
        ],
        axis=1,
    ).reshape(NW, NCHUNK, CHUNK)
    outc, outx = _gather_all(W_center, idx)
    return (outc[:, :, None], outx.reshape(B, C, EMB))


# speedup vs baseline: 1.4041x; 1.0263x over previous
import functools

import jax
import jax.numpy as jnp
from jax import lax
from jax.experimental import pallas as pl
from jax.experimental.pallas import tpu as pltpu
from jax.experimental.pallas import tpu_sc as plsc

VOCAB = 1000000
EMB = 64
B = 16384
C = 20
NC, NS = 2, 16
NW = NC * NS
BPW = B // NW
CHUNK = 128
KPB = BPW // CHUNK
NCHUNK = KPB * (1 + C)
NBUF = 4


def _gather_all(table, center2d, ctx3d):
    mesh = plsc.VectorSubcoreMesh(core_axis_name="c", subcore_axis_name="s",
                                  num_cores=NC, num_subcores=NS)

    @functools.partial(
        pl.kernel,
        out_type=(
            jax.ShapeDtypeStruct((B, EMB), jnp.float32),
            jax.ShapeDtypeStruct((C, B, EMB), jnp.float32),
        ),
        mesh=mesh,
        scratch_types=[
            pltpu.VMEM((NCHUNK, CHUNK), jnp.int32),
            pltpu.VMEM((NBUF, CHUNK, EMB), jnp.float32),
        ] + [pltpu.SemaphoreType.DMA] * (2 * NBUF),
        compiler_params=pltpu.CompilerParams(use_tc_tiling_on_sc=False),
    )
    def k(table_hbm, cen_hbm, ctx_hbm, outc_hbm, outx_hbm, idx_v, buf, *sems):
        gsem, wsem = sems[:NBUF], sems[NBUF:]
        wid = lax.axis_index("s") * NC + lax.axis_index("c")
        b0 = wid * BPW

        pltpu.sync_copy(cen_hbm.at[wid], idx_v.at[pl.ds(0, KPB)])
        for c in range(C):
            pltpu.sync_copy(ctx_hbm.at[c, wid],
                            idx_v.at[pl.ds(KPB * (1 + c), KPB)])

        def issue_gather(j, b):
            pltpu.async_copy(table_hbm.at[idx_v.at[j]], buf.at[b], gsem[b])

        def wait_gather(j, b):
            pltpu.make_async_copy(table_hbm.at[idx_v.at[j]], buf.at[b],
                                  gsem[b]).wait()

        def issue_wb(j, b):
            @pl.when(j < KPB)
            def _():
                pltpu.async_copy(buf.at[b],
                                 outc_hbm.at[pl.ds(b0 + j * CHUNK, CHUNK)],
                                 wsem[b])

            @pl.when(j >= KPB)
            def _():
                c = (j - KPB) // KPB
                kk = (j - KPB) % KPB
                pltpu.async_copy(
                    buf.at[b],
                    outx_hbm.at[c, pl.ds(b0 + kk * CHUNK, CHUNK)],
                    wsem[b])

        def wait_wb(b):
            pltpu.make_async_copy(buf.at[b], outc_hbm.at[pl.ds(0, CHUNK)],
                                  wsem[b]).wait()

        for b in range(NBUF):
            issue_gather(b, b)

        @pl.loop(0, NCHUNK - NBUF, step=NBUF)
        def _(j0):
            for b in range(NBUF):
                j = j0 + b
                wait_gather(j, b)
                issue_wb(j, b)
                wait_wb(b)
                issue_gather(j + NBUF, b)

        for b in range(NBUF):
            j = NCHUNK - NBUF + b
            wait_gather(j, b)
            issue_wb(j, b)
            wait_wb(b)

    return k(table, center2d, ctx3d)


def kernel(center_ids, context_ids, W_center, W_context):
    center2d = center_ids.astype(jnp.int32).reshape(NW, KPB, CHUNK)
    ctx3d = context_ids.astype(jnp.int32).T.reshape(C, NW, KPB, CHUNK)
    outc, outx = _gather_all(W_center, center2d, ctx3d)
    embs_center = outc[:, :, None]
    embs_context = outx.transpose(1, 0, 2)
    return (embs_center, embs_context)
